# baseline (speedup 1.0000x reference)
import functools

import jax
import jax.numpy as jnp
from jax import lax
from jax.experimental import pallas as pl
from jax.experimental.pallas import tpu as pltpu
from jax.experimental.pallas import tpu_sc as plsc

N = 10000
NP = 10240
E = 160000
EP = 163840
NC, NS, L = 2, 16, 16
NW = NC * NS
EW = EP // NW
CH = 128
NCHUNK = EW // CH
NPT = NP // NS
GAMMA = -0.1
ZETA = 1.1

f32 = jnp.float32
i32 = jnp.int32

def _mesh():
    return plsc.VectorSubcoreMesh(core_axis_name="c", subcore_axis_name="s",
                                  num_cores=NC, num_subcores=NS)


BN = 512
NBLK = NP // BN


def _dot(a, b):
    return jnp.dot(a, b, preferred_element_type=f32)


def _pre_math(xb, Wnb, bnb, Wself, bself, Wat, Wab, ba):
    hnb = jnp.maximum(_dot(xb, Wnb) + bnb, 0.0)
    hs = jnp.maximum(_dot(xb, Wself) + bself, 0.0)
    u = _dot(hnb, Wat) + ba
    v = _dot(hs, Wab)
    return u, v


def _pre0_body(x_ref, wnb_ref, bnb_ref, wself_ref, bself_ref, wat_ref,
               wab_ref, ba_ref, w_ref, u_ref, v_ref, xw_ref):
    xb = x_ref[...]
    u, v = _pre_math(xb, wnb_ref[...], bnb_ref[...], wself_ref[...],
                     bself_ref[...], wat_ref[...], wab_ref[...], ba_ref[...])
    u_ref[...] = u
    v_ref[...] = v
    xw_ref[...] = _dot(xb, w_ref[...])


def _full(shape):
    return pl.BlockSpec(shape, lambda i: (0, 0))


def _rows(width):
    return pl.BlockSpec((BN, width), lambda i: (i, 0))


def _tc_pre0(x, Wnb, bnb, Wself, bself, Wat, Wab, ba, W, din, hid):
    return pl.pallas_call(
        _pre0_body,
        grid=(NBLK,),
        in_specs=[
            _rows(din), _full((din, 64)), _full((1, 64)), _full((din, 64)),
            _full((1, 64)), _full((64, 16)), _full((64, 16)), _full((1, 16)),
            _full((din, hid)),
        ],
        out_specs=[_rows(16), _rows(16), _rows(hid)],
        out_shape=[
            jax.ShapeDtypeStruct((NP, 16), f32),
            jax.ShapeDtypeStruct((NP, 16), f32),
            jax.ShapeDtypeStruct((NP, hid), f32),
        ],
    )(x, Wnb, bnb, Wself, bself, Wat, Wab, ba, W)


def _dy_body(rs0_ref, rs1_ref, xw_ref, d_ref, *y_refs):
    rsum = rs0_ref[...] + rs1_ref[...] + 1.0
    d = jnp.minimum(jnp.maximum(lax.rsqrt(rsum), 0.0), 10.0)
    d_ref[...] = d
    y = d * xw_ref[...]
    for q, yq in enumerate(y_refs):
        yq[...] = y[:, q * 64:(q + 1) * 64]


def _tc_dy(rs0, rs1, xw, fdim):
    nq = fdim // 64
    return pl.pallas_call(
        _dy_body,
        grid=(NBLK,),
        in_specs=[_rows(1), _rows(1), _rows(fdim)],
        out_specs=[_rows(1)] + [_rows(64)] * nq,
        out_shape=[jax.ShapeDtypeStruct((NP, 1), f32)] +
                  [jax.ShapeDtypeStruct((NP, 64), f32)] * nq,
    )(rs0, rs1, xw)


def _mid_body(d_ref, s0, s1, s2, s3, y0, y1, y2, y3, wnb_ref, bnb_ref,
              wself_ref, bself_ref, wat_ref, wab_ref, ba_ref, w_ref,
              u_ref, v_ref, xw_ref):
    d = d_ref[...]
    parts = [jnp.maximum(d * (sr[...] + yr[...]), 0.0)
             for sr, yr in zip((s0, s1, s2, s3), (y0, y1, y2, y3))]
    xb = jnp.concatenate(parts, axis=1)
    u, v = _pre_math(xb, wnb_ref[...], bnb_ref[...], wself_ref[...],
                     bself_ref[...], wat_ref[...], wab_ref[...], ba_ref[...])
    u_ref[...] = u
    v_ref[...] = v
    xw_ref[...] = _dot(xb, w_ref[...])


def _tc_mid(d, ss, ys, Wnb, bnb, Wself, bself, Wat, Wab, ba, W):
    hid = 256
    return pl.pallas_call(
        _mid_body,
        grid=(NBLK,),
        in_specs=[_rows(1)] + [_rows(64)] * 8 + [
            _full((hid, 64)), _full((1, 64)), _full((hid, 64)),
            _full((1, 64)), _full((64, 16)), _full((64, 16)), _full((1, 16)),
            _full((hid, 128)),
        ],
        out_specs=[_rows(16), _rows(16), _rows(128)],
        out_shape=[
            jax.ShapeDtypeStruct((NP, 16), f32),
            jax.ShapeDtypeStruct((NP, 16), f32),
            jax.ShapeDtypeStruct((NP, 128), f32),
        ],
    )(d, *ss, *ys, Wnb, bnb, Wself, bself, Wat, Wab, ba, W)


def _final_body(d_ref, slo_ref, shi_ref, ylo_ref, yhi_ref, x2_ref):
    d = d_ref[...]
    x2lo = d * (slo_ref[...] + ylo_ref[...])
    x2hi = d * (shi_ref[...] + yhi_ref[...])
    x2_ref[...] = jnp.concatenate([x2lo, x2hi], axis=1)


def _tc_final(d, slo, shi, ylo, yhi):
    return pl.pallas_call(
        _final_body,
        grid=(NBLK,),
        in_specs=[_rows(1), _rows(64), _rows(64), _rows(64), _rows(64)],
        out_specs=_rows(128),
        out_shape=jax.ShapeDtypeStruct((NP, 128), f32),
    )(d, slo, shi, ylo, yhi)


DB = 1024


def _dec_body(a_ref, b_ref, o_ref):
    o_ref[...] = lax.dot_general(
        a_ref[...].astype(jnp.bfloat16), b_ref[...].astype(jnp.bfloat16),
        dimension_numbers=(((1,), (1,)), ((), ())),
        preferred_element_type=f32)


def _tc_decoder(x2p):
    return pl.pallas_call(
        _dec_body,
        grid=(NP // DB, NP // DB),
        in_specs=[
            pl.BlockSpec((DB, 128), lambda i, j: (i, 0)),
            pl.BlockSpec((DB, 128), lambda i, j: (j, 0)),
        ],
        out_specs=pl.BlockSpec((DB, DB), lambda i, j: (i, j)),
        out_shape=jax.ShapeDtypeStruct((N, N), f32),
        compiler_params=pltpu.CompilerParams(
            dimension_semantics=("parallel", "parallel")),
    )(x2p, x2p)


def _iota16():
    return lax.broadcasted_iota(i32, (L,), 0)


NCH1 = EW // CH


def _sc_phase1_body(row2d_hbm, col2d_hbm, u_hbm, v_hbm, wo_hbm, bo_hbm,
                    mask_hbm, rs_hbm,
                    row2d, col2d, maskall, u0b, u1b, v0b, v1b, wov, bov, zb,
                    racc, gu0, gu1, gv0, gv1, ss0, ss1):
    c = lax.axis_index("c")
    s = lax.axis_index("s")
    wid = c * NS + s
    base = wid * EW
    kbase = wid * NCH1

    for i in range(NPT // L):
        zb[pl.ds(i * L, L)] = jnp.zeros((L,), f32)
    pltpu.sync_copy(zb, racc.at[pl.ds(s * NPT, NPT)])

    pltpu.sync_copy(wo_hbm, wov)
    pltpu.sync_copy(bo_hbm, bov)
    pltpu.sync_copy(row2d_hbm.at[pl.ds(kbase, NCH1)], row2d)
    pltpu.sync_copy(col2d_hbm.at[pl.ds(kbase, NCH1)], col2d)
    plsc.subcore_barrier()

    ubufs, vbufs = [u0b, u1b], [v0b, v1b]
    gusems, gvsems = [gu0, gu1], [gv0, gv1]
    ssems = [ss0, ss1]

    def issue_gathers(k, b):
        pltpu.async_copy(u_hbm.at[row2d.at[k]], ubufs[b], gusems[b])
        pltpu.async_copy(v_hbm.at[col2d.at[k]], vbufs[b], gvsems[b])

    def wait_gathers(k, b):
        pltpu.make_async_copy(u_hbm.at[row2d.at[k]], ubufs[b],
                              gusems[b]).wait()
        pltpu.make_async_copy(v_hbm.at[col2d.at[k]], vbufs[b],
                              gvsems[b]).wait()

    def issue_scatter(k, b):
        pltpu.async_copy(maskall.at[pl.ds(k * CH, CH)],
                         racc.at[row2d.at[k]], ssems[b], add=True)

    def wait_scatter(k, b):
        pltpu.make_async_copy(maskall.at[pl.ds(k * CH, CH)],
                              racc.at[row2d.at[k]], ssems[b]).wait()

    issue_gathers(0, 0)
    issue_gathers(1, 1)

    def pair(p, carry):
        for b in range(2):
            k = p * 2 + b
            wait_gathers(k, b)
            for g in range(CH // L):
                ids = _iota16() + g * L
                acc = bov[...]
                for j in range(L):
                    jf = jnp.full((L,), j, i32)
                    uj = plsc.load_gather(ubufs[b], [ids, jf])
                    vj = plsc.load_gather(vbufs[b], [ids, jf])
                    acc = acc + jnp.maximum(uj + vj, 0.0) * wov[j, :]
                gate = 1.0 / (1.0 + jnp.exp(-acc))
                m = gate * (ZETA - GAMMA) + GAMMA
                m = jnp.minimum(jnp.maximum(m, 0.0), 1.0)
                maskall[pl.ds(k * CH + g * L, L)] = m
            nk = k + 2

            @pl.when(nk < NCH1)
            def _():
                issue_gathers(nk, b)

            @pl.when(k >= 2)
            def _():
                wait_scatter(k - 2, b)
            issue_scatter(k, b)
        return carry

    lax.fori_loop(0, NCH1 // 2, pair, 0)
    wait_scatter(NCH1 - 2, 0)
    wait_scatter(NCH1 - 1, 1)
    pltpu.sync_copy(maskall, mask_hbm.at[pl.ds(base, EW)])
    plsc.subcore_barrier()
    pltpu.sync_copy(racc.at[pl.ds(s * NPT, NPT)],
                    rs_hbm.at[pl.ds(c * NP + s * NPT, NPT)])


def _sc_phase1():
  return pl.kernel(
    _sc_phase1_body,
    out_type=[
        jax.ShapeDtypeStruct((EP,), f32),
        jax.ShapeDtypeStruct((2 * NP,), f32),
    ],
    mesh=_mesh(),
    scratch_types=[
        pltpu.VMEM((NCH1, CH), i32),
        pltpu.VMEM((NCH1, CH), i32),
        pltpu.VMEM((EW,), f32),
        pltpu.VMEM((CH, 16), f32),
        pltpu.VMEM((CH, 16), f32),
        pltpu.VMEM((CH, 16), f32),
        pltpu.VMEM((CH, 16), f32),
        pltpu.VMEM((16, 16), f32),
        pltpu.VMEM((16,), f32),
        pltpu.VMEM((NPT,), f32),
        pltpu.VMEM_SHARED((NP,), f32),
        pltpu.SemaphoreType.DMA,
        pltpu.SemaphoreType.DMA,
        pltpu.SemaphoreType.DMA,
        pltpu.SemaphoreType.DMA,
        pltpu.SemaphoreType.DMA,
        pltpu.SemaphoreType.DMA,
    ],
    compiler_params=pltpu.CompilerParams(needs_layout_passes=False, use_tc_tiling_on_sc=False),
  )


EW2 = EP // NS
CH2 = 256
NCH2 = EW2 // CH2


def _sc_phase2_body(fdim, row2d_hbm, col2d_hbm, mask_hbm, y2_hbm,
                    zero_hbm, s_hbm, row2d, col2d, maskv,
                    rows0, rows1, gs0, gs1, ss0, ss1, acc):
    c = lax.axis_index("c")
    s = lax.axis_index("s")
    base = s * EW2
    kbase = s * NCH2

    pltpu.sync_copy(zero_hbm, acc.at[pl.ds(s * NPT, NPT)])
    pltpu.sync_copy(row2d_hbm.at[pl.ds(kbase, NCH2)], row2d)
    pltpu.sync_copy(col2d_hbm.at[pl.ds(kbase, NCH2)], col2d)
    pltpu.sync_copy(mask_hbm.at[pl.ds(base, EW2)], maskv)

    coff = jnp.zeros((L,), i32) + c * NP

    def shift(k, carry):
        for g in range(CH2 // L):
            sl = pl.ds(g * L, L)
            col2d[k, sl] = col2d[k, sl] + coff
        return carry

    lax.fori_loop(0, NCH2, shift, 0)
    plsc.subcore_barrier()

    bufs = [rows0, rows1]
    gsems = [gs0, gs1]
    ssems = [ss0, ss1]

    def issue_gather(k, b):
        pltpu.async_copy(y2_hbm.at[col2d.at[k]], bufs[b], gsems[b])

    def wait_gather(k, b):
        pltpu.make_async_copy(y2_hbm.at[col2d.at[k]], bufs[b],
                              gsems[b]).wait()

    def issue_scatter(k, b):
        pltpu.async_copy(bufs[b], acc.at[row2d.at[k]], ssems[b], add=True)

    def wait_scatter(k, b):
        pltpu.make_async_copy(bufs[b], acc.at[row2d.at[k]], ssems[b]).wait()

    issue_gather(0, 0)
    issue_gather(1, 1)

    def pair(p, carry):
        for b in range(2):
            k = p * 2 + b
            wait_gather(k, b)

            @plsc.parallel_loop(0, CH2, step=1, unroll=4)
            def _(e):
                msk = plsc.load_gather(
                    maskv, [jnp.zeros((L,), i32) + (k * CH2 + e)])
                for f in range(fdim // L):
                    sl = pl.ds(f * L, L)
                    bufs[b][e, sl] = bufs[b][e, sl] * msk

            issue_scatter(k, b)
            nk = k + 2

            @pl.when(nk < NCH2)
            def _():
                wait_scatter(k, b)
                issue_gather(nk, b)
        return carry

    lax.fori_loop(0, NCH2 // 2, pair, 0)
    wait_scatter(NCH2 - 2, 0)
    wait_scatter(NCH2 - 1, 1)
    plsc.subcore_barrier()
    pltpu.sync_copy(acc.at[pl.ds(s * NPT, NPT)],
                    s_hbm.at[pl.ds(c * NP + s * NPT, NPT)])


def _make_sc_phase2(fdim):
    return pl.kernel(
        functools.partial(_sc_phase2_body, fdim),
        out_type=jax.ShapeDtypeStruct((2 * NP, fdim), f32),
        mesh=_mesh(),
        scratch_types=[
            pltpu.VMEM((NCH2, CH2), i32),
            pltpu.VMEM((NCH2, CH2), i32),
            pltpu.VMEM((EW2,), f32),
            pltpu.VMEM((CH2, fdim), f32),
            pltpu.VMEM((CH2, fdim), f32),
            pltpu.SemaphoreType.DMA,
            pltpu.SemaphoreType.DMA,
            pltpu.SemaphoreType.DMA,
            pltpu.SemaphoreType.DMA,
            pltpu.VMEM_SHARED((NP, fdim), f32),
        ],
        compiler_params=pltpu.CompilerParams(needs_layout_passes=False, use_tc_tiling_on_sc=False),
    )


def kernel(x, edge_index, W0, W1, Wnb0, bnb0, Wself0, bself0, Wa0, ba0, Wo0,
           bo0, Wnb1, bnb1, Wself1, bself1, Wa1, ba1, Wo1, bo1):
    xp = jnp.pad(x, ((0, NP - N), (0, 0)))
    pad_idx = jnp.full((EP - E,), NP - 1, i32)
    rowp = jnp.concatenate([edge_index[0], pad_idx])
    colp = jnp.concatenate([edge_index[1], pad_idx])

    def layer(row, col, Wo, bo, u, v, xw, fdim):
        wosq = jnp.broadcast_to(Wo, (16, 16))
        bo16 = jnp.broadcast_to(bo, (16,))
        row2d = row.reshape(EP // CH, CH)
        col2d = col.reshape(EP // CH, CH)
        row2d2 = row.reshape(EP // CH2, CH2)
        col2d2 = col.reshape(EP // CH2, CH2)
        mask, rs = _sc_phase1()(row2d, col2d, u, v, wosq, bo16)
        rs0 = rs[:NP].reshape(NP, 1)
        rs1 = rs[NP:].reshape(NP, 1)
        d, *yq = _tc_dy(rs0, rs1, xw, fdim)
        zero = jnp.zeros((NPT, 64), f32)
        sq = []
        for p in range(fdim // 128):
            y2 = jnp.concatenate([yq[2 * p], yq[2 * p + 1]], axis=0)
            sflat = _make_sc_phase2(64)(row2d2, col2d2, mask, y2, zero)
            sq.extend([sflat[:NP], sflat[NP:]])
        return d, sq, yq

    r2 = lambda b: b.reshape(1, -1)

    u0, v0, xw0 = _tc_pre0(xp, Wnb0, r2(bnb0), Wself0, r2(bself0),
                           Wa0[:64], Wa0[64:], r2(ba0), W0, 256, 256)
    d0, s0q, y0q = layer(rowp, colp, Wo0, bo0, u0, v0, xw0, 256)
    u1, v1, xw1 = _tc_mid(d0, s0q, y0q, Wnb1, r2(bnb1),
                          Wself1, r2(bself1), Wa1[:64], Wa1[64:], r2(ba1), W1)
    d1, s1q, y1q = layer(rowp, colp, Wo1, bo1, u1, v1, xw1, 128)
    x2p = _tc_final(d1, s1q[0], s1q[1], y1q[0], y1q[1])
    pred = _tc_decoder(x2p)
    return (x2p[:N], pred.reshape(-1))


# speedup vs baseline: 1.0006x; 1.0006x over previous
import functools

import jax
import jax.numpy as jnp
from jax import lax
from jax.experimental import pallas as pl
from jax.experimental.pallas import tpu as pltpu
from jax.experimental.pallas import tpu_sc as plsc

N = 10000
NP = 10240
E = 160000
EP = 163840
NC, NS, L = 2, 16, 16
NW = NC * NS
EW = EP // NW
CH = 128
NCHUNK = EW // CH
NPT = NP // NS
GAMMA = -0.1
ZETA = 1.1

f32 = jnp.float32
i32 = jnp.int32

def _mesh():
    return plsc.VectorSubcoreMesh(core_axis_name="c", subcore_axis_name="s",
                                  num_cores=NC, num_subcores=NS)


BN = 512
NBLK = NP // BN


def _dot(a, b):
    return jnp.dot(a, b, preferred_element_type=f32)


def _pre_math(xb, Wnb, bnb, Wself, bself, Wat, Wab, ba):
    hnb = jnp.maximum(_dot(xb, Wnb) + bnb, 0.0)
    hs = jnp.maximum(_dot(xb, Wself) + bself, 0.0)
    u = _dot(hnb, Wat) + ba
    v = _dot(hs, Wab)
    return u, v


def _pre0_body(x_ref, wnb_ref, bnb_ref, wself_ref, bself_ref, wat_ref,
               wab_ref, ba_ref, w_ref, u_ref, v_ref, xw_ref):
    xb = x_ref[...]
    u, v = _pre_math(xb, wnb_ref[...], bnb_ref[...], wself_ref[...],
                     bself_ref[...], wat_ref[...], wab_ref[...], ba_ref[...])
    u_ref[...] = u
    v_ref[...] = v
    xw_ref[...] = _dot(xb, w_ref[...])


def _full(shape):
    return pl.BlockSpec(shape, lambda i: (0, 0))


def _rows(width):
    return pl.BlockSpec((BN, width), lambda i: (i, 0))


def _tc_pre0(x, Wnb, bnb, Wself, bself, Wat, Wab, ba, W, din, hid):
    return pl.pallas_call(
        _pre0_body,
        grid=(NBLK,),
        in_specs=[
            _rows(din), _full((din, 64)), _full((1, 64)), _full((din, 64)),
            _full((1, 64)), _full((64, 16)), _full((64, 16)), _full((1, 16)),
            _full((din, hid)),
        ],
        out_specs=[_rows(16), _rows(16), _rows(hid)],
        out_shape=[
            jax.ShapeDtypeStruct((NP, 16), f32),
            jax.ShapeDtypeStruct((NP, 16), f32),
            jax.ShapeDtypeStruct((NP, hid), f32),
        ],
    )(x, Wnb, bnb, Wself, bself, Wat, Wab, ba, W)


def _dy_body(rs0_ref, rs1_ref, xw_ref, d_ref, *y_refs):
    rsum = rs0_ref[...] + rs1_ref[...] + 1.0
    d = jnp.minimum(jnp.maximum(lax.rsqrt(rsum), 0.0), 10.0)
    d_ref[...] = d
    y = d * xw_ref[...]
    for q, yq in enumerate(y_refs):
        yq[...] = y[:, q * 64:(q + 1) * 64]


def _tc_dy(rs0, rs1, xw, fdim):
    nq = fdim // 64
    return pl.pallas_call(
        _dy_body,
        grid=(NBLK,),
        in_specs=[_rows(1), _rows(1), _rows(fdim)],
        out_specs=[_rows(1)] + [_rows(64)] * nq,
        out_shape=[jax.ShapeDtypeStruct((NP, 1), f32)] +
                  [jax.ShapeDtypeStruct((NP, 64), f32)] * nq,
    )(rs0, rs1, xw)


def _mid_body(d_ref, s0, s1, s2, s3, y0, y1, y2, y3, wnb_ref, bnb_ref,
              wself_ref, bself_ref, wat_ref, wab_ref, ba_ref, w_ref,
              u_ref, v_ref, xw_ref):
    d = d_ref[...]
    parts = [jnp.maximum(d * (sr[...] + yr[...]), 0.0)
             for sr, yr in zip((s0, s1, s2, s3), (y0, y1, y2, y3))]
    xb = jnp.concatenate(parts, axis=1)
    u, v = _pre_math(xb, wnb_ref[...], bnb_ref[...], wself_ref[...],
                     bself_ref[...], wat_ref[...], wab_ref[...], ba_ref[...])
    u_ref[...] = u
    v_ref[...] = v
    xw_ref[...] = _dot(xb, w_ref[...])


def _tc_mid(d, ss, ys, Wnb, bnb, Wself, bself, Wat, Wab, ba, W):
    hid = 256
    return pl.pallas_call(
        _mid_body,
        grid=(NBLK,),
        in_specs=[_rows(1)] + [_rows(64)] * 8 + [
            _full((hid, 64)), _full((1, 64)), _full((hid, 64)),
            _full((1, 64)), _full((64, 16)), _full((64, 16)), _full((1, 16)),
            _full((hid, 128)),
        ],
        out_specs=[_rows(16), _rows(16), _rows(128)],
        out_shape=[
            jax.ShapeDtypeStruct((NP, 16), f32),
            jax.ShapeDtypeStruct((NP, 16), f32),
            jax.ShapeDtypeStruct((NP, 128), f32),
        ],
    )(d, *ss, *ys, Wnb, bnb, Wself, bself, Wat, Wab, ba, W)


def _final_body(d_ref, slo_ref, shi_ref, ylo_ref, yhi_ref, x2_ref):
    d = d_ref[...]
    x2lo = d * (slo_ref[...] + ylo_ref[...])
    x2hi = d * (shi_ref[...] + yhi_ref[...])
    x2_ref[...] = jnp.concatenate([x2lo, x2hi], axis=1)


def _tc_final(d, slo, shi, ylo, yhi):
    return pl.pallas_call(
        _final_body,
        grid=(NBLK,),
        in_specs=[_rows(1), _rows(64), _rows(64), _rows(64), _rows(64)],
        out_specs=_rows(128),
        out_shape=jax.ShapeDtypeStruct((NP, 128), f32),
    )(d, slo, shi, ylo, yhi)


DB = 1024


def _dec_body(a_ref, b_ref, o_ref):
    o_ref[...] = lax.dot_general(
        a_ref[...], b_ref[...],
        dimension_numbers=(((1,), (1,)), ((), ())),
        preferred_element_type=f32)


def _tc_decoder(x2p):
    return pl.pallas_call(
        _dec_body,
        grid=(NP // DB, NP // DB),
        in_specs=[
            pl.BlockSpec((DB, 128), lambda i, j: (i, 0)),
            pl.BlockSpec((DB, 128), lambda i, j: (j, 0)),
        ],
        out_specs=pl.BlockSpec((DB, DB), lambda i, j: (i, j)),
        out_shape=jax.ShapeDtypeStruct((N, N), f32),
        compiler_params=pltpu.CompilerParams(
            dimension_semantics=("parallel", "parallel")),
    )(x2p, x2p)


def _iota16():
    return lax.broadcasted_iota(i32, (L,), 0)


NCH1 = EW // CH


def _sc_phase1_body(row2d_hbm, col2d_hbm, u_hbm, v_hbm, wo_hbm, bo_hbm,
                    mask_hbm, rs_hbm,
                    row2d, col2d, maskall, u0b, u1b, v0b, v1b, wov, bov, zb,
                    racc, gu0, gu1, gv0, gv1, ss0, ss1):
    c = lax.axis_index("c")
    s = lax.axis_index("s")
    wid = c * NS + s
    base = wid * EW
    kbase = wid * NCH1

    for i in range(NPT // L):
        zb[pl.ds(i * L, L)] = jnp.zeros((L,), f32)
    pltpu.sync_copy(zb, racc.at[pl.ds(s * NPT, NPT)])

    pltpu.sync_copy(wo_hbm, wov)
    pltpu.sync_copy(bo_hbm, bov)
    pltpu.sync_copy(row2d_hbm.at[pl.ds(kbase, NCH1)], row2d)
    pltpu.sync_copy(col2d_hbm.at[pl.ds(kbase, NCH1)], col2d)
    plsc.subcore_barrier()

    ubufs, vbufs = [u0b, u1b], [v0b, v1b]
    gusems, gvsems = [gu0, gu1], [gv0, gv1]
    ssems = [ss0, ss1]

    def issue_gathers(k, b):
        pltpu.async_copy(u_hbm.at[row2d.at[k]], ubufs[b], gusems[b])
        pltpu.async_copy(v_hbm.at[col2d.at[k]], vbufs[b], gvsems[b])

    def wait_gathers(k, b):
        pltpu.make_async_copy(u_hbm.at[row2d.at[k]], ubufs[b],
                              gusems[b]).wait()
        pltpu.make_async_copy(v_hbm.at[col2d.at[k]], vbufs[b],
                              gvsems[b]).wait()

    def issue_scatter(k, b):
        pltpu.async_copy(maskall.at[pl.ds(k * CH, CH)],
                         racc.at[row2d.at[k]], ssems[b], add=True)

    def wait_scatter(k, b):
        pltpu.make_async_copy(maskall.at[pl.ds(k * CH, CH)],
                              racc.at[row2d.at[k]], ssems[b]).wait()

    issue_gathers(0, 0)
    issue_gathers(1, 1)

    def pair(p, carry):
        for b in range(2):
            k = p * 2 + b
            wait_gathers(k, b)
            for g in range(CH // L):
                ids = _iota16() + g * L
                acc = bov[...]
                for j in range(L):
                    jf = jnp.full((L,), j, i32)
                    uj = plsc.load_gather(ubufs[b], [ids, jf])
                    vj = plsc.load_gather(vbufs[b], [ids, jf])
                    acc = acc + jnp.maximum(uj + vj, 0.0) * wov[j, :]
                gate = 1.0 / (1.0 + jnp.exp(-acc))
                m = gate * (ZETA - GAMMA) + GAMMA
                m = jnp.minimum(jnp.maximum(m, 0.0), 1.0)
                maskall[pl.ds(k * CH + g * L, L)] = m
            nk = k + 2

            @pl.when(nk < NCH1)
            def _():
                issue_gathers(nk, b)

            @pl.when(k >= 2)
            def _():
                wait_scatter(k - 2, b)
            issue_scatter(k, b)
        return carry

    lax.fori_loop(0, NCH1 // 2, pair, 0)
    wait_scatter(NCH1 - 2, 0)
    wait_scatter(NCH1 - 1, 1)
    pltpu.sync_copy(maskall, mask_hbm.at[pl.ds(base, EW)])
    plsc.subcore_barrier()
    pltpu.sync_copy(racc.at[pl.ds(s * NPT, NPT)],
                    rs_hbm.at[pl.ds(c * NP + s * NPT, NPT)])


def _sc_phase1():
  return pl.kernel(
    _sc_phase1_body,
    out_type=[
        jax.ShapeDtypeStruct((EP,), f32),
        jax.ShapeDtypeStruct((2 * NP,), f32),
    ],
    mesh=_mesh(),
    scratch_types=[
        pltpu.VMEM((NCH1, CH), i32),
        pltpu.VMEM((NCH1, CH), i32),
        pltpu.VMEM((EW,), f32),
        pltpu.VMEM((CH, 16), f32),
        pltpu.VMEM((CH, 16), f32),
        pltpu.VMEM((CH, 16), f32),
        pltpu.VMEM((CH, 16), f32),
        pltpu.VMEM((16, 16), f32),
        pltpu.VMEM((16,), f32),
        pltpu.VMEM((NPT,), f32),
        pltpu.VMEM_SHARED((NP,), f32),
        pltpu.SemaphoreType.DMA,
        pltpu.SemaphoreType.DMA,
        pltpu.SemaphoreType.DMA,
        pltpu.SemaphoreType.DMA,
        pltpu.SemaphoreType.DMA,
        pltpu.SemaphoreType.DMA,
    ],
    compiler_params=pltpu.CompilerParams(needs_layout_passes=False, use_tc_tiling_on_sc=False),
  )


EW2 = EP // NS
CH2 = 256
NCH2 = EW2 // CH2


def _sc_phase2_body(fdim, row2d_hbm, col2d_hbm, mask_hbm, y2_hbm,
                    zero_hbm, s_hbm, row2d, col2d, maskv,
                    rows0, rows1, gs0, gs1, ss0, ss1, acc):
    c = lax.axis_index("c")
    s = lax.axis_index("s")
    base = s * EW2
    kbase = s * NCH2

    pltpu.sync_copy(zero_hbm, acc.at[pl.ds(s * NPT, NPT)])
    pltpu.sync_copy(row2d_hbm.at[pl.ds(kbase, NCH2)], row2d)
    pltpu.sync_copy(col2d_hbm.at[pl.ds(kbase, NCH2)], col2d)
    pltpu.sync_copy(mask_hbm.at[pl.ds(base, EW2)], maskv)

    coff = jnp.zeros((L,), i32) + c * NP

    def shift(k, carry):
        for g in range(CH2 // L):
            sl = pl.ds(g * L, L)
            col2d[k, sl] = col2d[k, sl] + coff
        return carry

    lax.fori_loop(0, NCH2, shift, 0)
    plsc.subcore_barrier()

    bufs = [rows0, rows1]
    gsems = [gs0, gs1]
    ssems = [ss0, ss1]

    def issue_gather(k, b):
        pltpu.async_copy(y2_hbm.at[col2d.at[k]], bufs[b], gsems[b])

    def wait_gather(k, b):
        pltpu.make_async_copy(y2_hbm.at[col2d.at[k]], bufs[b],
                              gsems[b]).wait()

    def issue_scatter(k, b):
        pltpu.async_copy(bufs[b], acc.at[row2d.at[k]], ssems[b], add=True)

    def wait_scatter(k, b):
        pltpu.make_async_copy(bufs[b], acc.at[row2d.at[k]], ssems[b]).wait()

    issue_gather(0, 0)
    issue_gather(1, 1)

    def pair(p, carry):
        for b in range(2):
            k = p * 2 + b
            wait_gather(k, b)

            @plsc.parallel_loop(0, CH2, step=1, unroll=4)
            def _(e):
                msk = plsc.load_gather(
                    maskv, [jnp.zeros((L,), i32) + (k * CH2 + e)])
                for f in range(fdim // L):
                    sl = pl.ds(f * L, L)
                    bufs[b][e, sl] = bufs[b][e, sl] * msk

            issue_scatter(k, b)
            nk = k + 2

            @pl.when(nk < NCH2)
            def _():
                wait_scatter(k, b)
                issue_gather(nk, b)
        return carry

    lax.fori_loop(0, NCH2 // 2, pair, 0)
    wait_scatter(NCH2 - 2, 0)
    wait_scatter(NCH2 - 1, 1)
    plsc.subcore_barrier()
    pltpu.sync_copy(acc.at[pl.ds(s * NPT, NPT)],
                    s_hbm.at[pl.ds(c * NP + s * NPT, NPT)])


def _make_sc_phase2(fdim):
    return pl.kernel(
        functools.partial(_sc_phase2_body, fdim),
        out_type=jax.ShapeDtypeStruct((2 * NP, fdim), f32),
        mesh=_mesh(),
        scratch_types=[
            pltpu.VMEM((NCH2, CH2), i32),
            pltpu.VMEM((NCH2, CH2), i32),
            pltpu.VMEM((EW2,), f32),
            pltpu.VMEM((CH2, fdim), f32),
            pltpu.VMEM((CH2, fdim), f32),
            pltpu.SemaphoreType.DMA,
            pltpu.SemaphoreType.DMA,
            pltpu.SemaphoreType.DMA,
            pltpu.SemaphoreType.DMA,
            pltpu.VMEM_SHARED((NP, fdim), f32),
        ],
        compiler_params=pltpu.CompilerParams(needs_layout_passes=False, use_tc_tiling_on_sc=False),
    )


def kernel(x, edge_index, W0, W1, Wnb0, bnb0, Wself0, bself0, Wa0, ba0, Wo0,
           bo0, Wnb1, bnb1, Wself1, bself1, Wa1, ba1, Wo1, bo1):
    xp = jnp.pad(x, ((0, NP - N), (0, 0)))
    pad_idx = jnp.full((EP - E,), NP - 1, i32)
    rowp = jnp.concatenate([edge_index[0], pad_idx])
    colp = jnp.concatenate([edge_index[1], pad_idx])

    def layer(row, col, Wo, bo, u, v, xw, fdim):
        wosq = jnp.broadcast_to(Wo, (16, 16))
        bo16 = jnp.broadcast_to(bo, (16,))
        row2d = row.reshape(EP // CH, CH)
        col2d = col.reshape(EP // CH, CH)
        row2d2 = row.reshape(EP // CH2, CH2)
        col2d2 = col.reshape(EP // CH2, CH2)
        mask, rs = _sc_phase1()(row2d, col2d, u, v, wosq, bo16)
        rs0 = rs[:NP].reshape(NP, 1)
        rs1 = rs[NP:].reshape(NP, 1)
        d, *yq = _tc_dy(rs0, rs1, xw, fdim)
        zero = jnp.zeros((NPT, 64), f32)
        sq = []
        for p in range(fdim // 128):
            y2 = jnp.concatenate([yq[2 * p], yq[2 * p + 1]], axis=0)
            sflat = _make_sc_phase2(64)(row2d2, col2d2, mask, y2, zero)
            sq.extend([sflat[:NP], sflat[NP:]])
        return d, sq, yq

    r2 = lambda b: b.reshape(1, -1)

    u0, v0, xw0 = _tc_pre0(xp, Wnb0, r2(bnb0), Wself0, r2(bself0),
                           Wa0[:64], Wa0[64:], r2(ba0), W0, 256, 256)
    d0, s0q, y0q = layer(rowp, colp, Wo0, bo0, u0, v0, xw0, 256)
    u1, v1, xw1 = _tc_mid(d0, s0q, y0q, Wnb1, r2(bnb1),
                          Wself1, r2(bself1), Wa1[:64], Wa1[64:], r2(ba1), W1)
    d1, s1q, y1q = layer(rowp, colp, Wo1, bo1, u1, v1, xw1, 128)
    x2p = _tc_final(d1, s1q[0], s1q[1], y1q[0], y1q[1])
    pred = _tc_decoder(x2p)
    return (x2p[:N], pred.reshape(-1))
